# baseline (speedup 1.0000x reference)
import jax
import jax.numpy as jnp
from jax.experimental import pallas as pl
from jax.experimental.pallas import tpu as pltpu

N_I_ = 4096
N_J_ = 8192
CH = 256
M_BLK = 256
N_STEPS = N_I_ // M_BLK
GII_BUFS = 2
GJI_BUFS = 3
O_BUFS = 2


def _body(xi_h, gii_h, xj_h, gji_h, w1_h, b1_h, w2_h, b2_h, out_h,
          xi_s, xj_s, w1_s, b1_s, w2_s, b2_s, yi_s, yj_s,
          gii_b, gji_b, out_b,
          sem_x, sem_gii, sem_gji, sem_out):
    bf16 = jnp.bfloat16

    half = M_BLK // 2

    def gii_copy(step, h):
        return pltpu.make_async_copy(
            gii_h.at[pl.ds(step * M_BLK + h * half, half), :],
            gii_b.at[step % GII_BUFS, pl.ds(h * half, half), :],
            sem_gii.at[step % GII_BUFS, h])

    def gji_copy(step, h):
        return pltpu.make_async_copy(
            gji_h.at[pl.ds(step * M_BLK + h * half, half), :],
            gji_b.at[step % GJI_BUFS, pl.ds(h * half, half), :],
            sem_gji.at[step % GJI_BUFS, h])

    def out_copy(step):
        return pltpu.make_async_copy(
            out_b.at[step % O_BUFS],
            out_h.at[pl.ds(step * M_BLK, M_BLK), :], sem_out.at[step % O_BUFS])

    x_copies = [
        pltpu.make_async_copy(xi_h, xi_s, sem_x),
        pltpu.make_async_copy(xj_h, xj_s, sem_x),
        pltpu.make_async_copy(w1_h, w1_s, sem_x),
        pltpu.make_async_copy(b1_h, b1_s, sem_x),
        pltpu.make_async_copy(w2_h, w2_s, sem_x),
        pltpu.make_async_copy(b2_h, b2_s, sem_x),
    ]
    for c in x_copies:
        c.start()
    for s in range(GII_BUFS):
        gii_copy(s, 0).start()
        gii_copy(s, 1).start()
    for s in range(GJI_BUFS):
        gji_copy(s, 0).start()
        gji_copy(s, 1).start()
    for c in x_copies:
        c.wait()

    yi_s[...] = (jnp.dot(xi_s[...].astype(bf16), w1_s[...].astype(bf16),
                         preferred_element_type=jnp.float32)
                 + b1_s[...]).astype(bf16)
    yj_s[...] = (jnp.dot(xj_s[...].astype(bf16), w2_s[...].astype(bf16),
                         preferred_element_type=jnp.float32)
                 + b2_s[...]).astype(bf16)

    for i in range(N_STEPS):
        gii_copy(i, 0).wait()
        gii_copy(i, 1).wait()
        gji_copy(i, 0).wait()
        gji_copy(i, 1).wait()
        acc = jnp.dot(gii_b[i % GII_BUFS].astype(bf16), yi_s[...],
                      preferred_element_type=jnp.float32)
        acc = acc + jnp.dot(gji_b[i % GJI_BUFS].astype(bf16), yj_s[...],
                            preferred_element_type=jnp.float32)
        if i >= O_BUFS:
            out_copy(i - O_BUFS).wait()
        out_b[i % O_BUFS] = jnp.maximum(acc, 0.0)
        out_copy(i).start()
        if i + GII_BUFS < N_STEPS:
            gii_copy(i + GII_BUFS, 0).start()
            gii_copy(i + GII_BUFS, 1).start()
        if i + GJI_BUFS < N_STEPS:
            gji_copy(i + GJI_BUFS, 0).start()
            gji_copy(i + GJI_BUFS, 1).start()

    for i in range(N_STEPS - O_BUFS, N_STEPS):
        out_copy(i).wait()


def kernel(xi, Gi2i, xj, Gj2i, W1, b1, W2, b2):
    n_i = Gi2i.shape[0]
    n_j = xj.shape[0]
    any_spec = pl.BlockSpec(memory_space=pl.ANY)
    return pl.pallas_call(
        _body,
        in_specs=[any_spec] * 8,
        out_specs=any_spec,
        out_shape=jax.ShapeDtypeStruct((n_i, CH), jnp.float32),
        scratch_shapes=[
            pltpu.VMEM((n_i, CH), jnp.float32),
            pltpu.VMEM((n_j, CH), jnp.float32),
            pltpu.VMEM((CH, CH), jnp.float32),
            pltpu.VMEM((1, CH), jnp.float32),
            pltpu.VMEM((CH, CH), jnp.float32),
            pltpu.VMEM((1, CH), jnp.float32),
            pltpu.VMEM((n_i, CH), jnp.bfloat16),
            pltpu.VMEM((n_j, CH), jnp.bfloat16),
            pltpu.VMEM((GII_BUFS, M_BLK, n_i), jnp.float32),
            pltpu.VMEM((GJI_BUFS, M_BLK, n_j), jnp.float32),
            pltpu.VMEM((O_BUFS, M_BLK, CH), jnp.float32),
            pltpu.SemaphoreType.DMA,
            pltpu.SemaphoreType.DMA((GII_BUFS, 2)),
            pltpu.SemaphoreType.DMA((GJI_BUFS, 2)),
            pltpu.SemaphoreType.DMA((O_BUFS,)),
        ],
    )(xi, Gi2i, xj, Gj2i, W1, b1.reshape(1, CH), W2, b2.reshape(1, CH))


# speedup vs baseline: 1.0575x; 1.0575x over previous
import jax
import jax.numpy as jnp
from jax.experimental import pallas as pl
from jax.experimental.pallas import tpu as pltpu

N_I_ = 4096
N_J_ = 8192
CH = 256
M_BLK = 256


def _body(xi_ref, gii_ref, xj_ref, gji_ref, w1_ref, b1_ref, w2_ref, b2_ref,
          out_ref, yi_ref, yj_ref):
    i = pl.program_id(0)

    @pl.when(i == 0)
    def _prologue():
        yi = jnp.dot(xi_ref[...].astype(jnp.bfloat16),
                     w1_ref[...].astype(jnp.bfloat16),
                     preferred_element_type=jnp.float32) + b1_ref[...]
        yi_ref[...] = yi.astype(jnp.bfloat16)
        yj = jnp.dot(xj_ref[...].astype(jnp.bfloat16),
                     w2_ref[...].astype(jnp.bfloat16),
                     preferred_element_type=jnp.float32) + b2_ref[...]
        yj_ref[...] = yj.astype(jnp.bfloat16)

    acc = jnp.dot(gii_ref[...].astype(jnp.bfloat16), yi_ref[...],
                  preferred_element_type=jnp.float32)
    acc = acc + jnp.dot(gji_ref[...].astype(jnp.bfloat16), yj_ref[...],
                        preferred_element_type=jnp.float32)
    out_ref[...] = jnp.maximum(acc, 0.0)


def kernel(xi, Gi2i, xj, Gj2i, W1, b1, W2, b2):
    n_i = Gi2i.shape[0]
    n_j = xj.shape[0]
    grid = (n_i // M_BLK,)
    return pl.pallas_call(
        _body,
        grid=grid,
        in_specs=[
            pl.BlockSpec((n_i, CH), lambda i: (0, 0)),
            pl.BlockSpec((M_BLK, n_i), lambda i: (i, 0)),
            pl.BlockSpec((n_j, CH), lambda i: (0, 0)),
            pl.BlockSpec((M_BLK, n_j), lambda i: (i, 0)),
            pl.BlockSpec((CH, CH), lambda i: (0, 0)),
            pl.BlockSpec((1, CH), lambda i: (0, 0)),
            pl.BlockSpec((CH, CH), lambda i: (0, 0)),
            pl.BlockSpec((1, CH), lambda i: (0, 0)),
        ],
        out_specs=pl.BlockSpec((M_BLK, CH), lambda i: (i, 0)),
        out_shape=jax.ShapeDtypeStruct((n_i, CH), jnp.float32),
        scratch_shapes=[
            pltpu.VMEM((n_i, CH), jnp.bfloat16),
            pltpu.VMEM((n_j, CH), jnp.bfloat16),
        ],
    )(xi, Gi2i, xj, Gj2i, W1, b1.reshape(1, CH), W2, b2.reshape(1, CH))
